# Initial kernel scaffold; baseline (speedup 1.0000x reference)
import functools

import jax
import jax.numpy as jnp
from jax import lax
from jax.experimental import pallas as pl
from jax.experimental.pallas import tpu as pltpu
from jax.experimental.pallas import tpu_sc as plsc

B = 16384
EMB_DIM = 32

_info = plsc.get_sparse_core_info()
_NC, _NS = _info.num_cores, _info.num_subcores
_NW = _NC * _NS
_BPW = B // _NW


def _make_gather():
    mesh = plsc.VectorSubcoreMesh(core_axis_name="c", subcore_axis_name="s")

    @functools.partial(
        pl.kernel,
        mesh=mesh,
        out_type=(
            jax.ShapeDtypeStruct((B, EMB_DIM), jnp.float32),
            jax.ShapeDtypeStruct((B, EMB_DIM), jnp.float32),
        ),
        scratch_types=[
            pltpu.VMEM((_BPW,), jnp.int32),
            pltpu.VMEM((_BPW, EMB_DIM), jnp.float32),
            pltpu.VMEM((_BPW,), jnp.int32),
            pltpu.VMEM((_BPW, EMB_DIM), jnp.float32),
            pltpu.SemaphoreType.DMA,
            pltpu.SemaphoreType.DMA,
        ],
    )
    def gather_k(ut_hbm, uid_hbm, ft_hbm, fid_hbm, out_u, out_f,
                 uidx_v, urows_v, fidx_v, frows_v, usem, fsem):
        wid = lax.axis_index("s") * _NC + lax.axis_index("c")
        base = wid * _BPW
        pltpu.sync_copy(uid_hbm.at[pl.ds(base, _BPW)], uidx_v)
        pltpu.sync_copy(fid_hbm.at[pl.ds(base, _BPW)], fidx_v)
        cu = pltpu.async_copy(ut_hbm.at[uidx_v], urows_v, usem)
        cf = pltpu.async_copy(ft_hbm.at[fidx_v], frows_v, fsem)
        cu.wait()
        cf.wait()
        pltpu.sync_copy(urows_v, out_u.at[pl.ds(base, _BPW)])
        pltpu.sync_copy(frows_v, out_f.at[pl.ds(base, _BPW)])

    return gather_k


_gather = _make_gather()

_BM = 1024


def _mlp_body(xu_ref, xf_ref, w1_ref, b1_ref, w2_ref, b2_ref, w3_ref, b3_ref,
              out_ref):
    x = jnp.concatenate([xu_ref[...], xf_ref[...]], axis=1)
    h = jnp.dot(x, w1_ref[...], preferred_element_type=jnp.float32)
    h = jnp.maximum(h + b1_ref[...], 0.0)
    h = jnp.dot(h, w2_ref[...], preferred_element_type=jnp.float32)
    h = jnp.maximum(h + b2_ref[...], 0.0)
    out_ref[...] = (
        jnp.dot(h, w3_ref[...], preferred_element_type=jnp.float32)
        + b3_ref[...]
    )


def _mlp(xu, xf, W1, b1, W2, b2, W3, b3):
    d1, d2, d3 = W1.shape[1], W2.shape[1], W3.shape[1]
    grid = (B // _BM,)

    def full(shape):
        return pl.BlockSpec(shape, lambda i: (0, 0))

    return pl.pallas_call(
        _mlp_body,
        grid=grid,
        in_specs=[
            pl.BlockSpec((_BM, EMB_DIM), lambda i: (i, 0)),
            pl.BlockSpec((_BM, EMB_DIM), lambda i: (i, 0)),
            full(W1.shape),
            full((1, d1)),
            full(W2.shape),
            full((1, d2)),
            full(W3.shape),
            full((1, d3)),
        ],
        out_specs=pl.BlockSpec((_BM, d3), lambda i: (i, 0)),
        out_shape=jax.ShapeDtypeStruct((B, d3), jnp.float32),
    )(xu, xf, W1, b1.reshape(1, d1), W2, b2.reshape(1, d2), W3,
      b3.reshape(1, d3))


def kernel(user_ids, emotion_ids, user_table, feeling_table,
           W1, b1, W2, b2, W3, b3):
    uid = user_ids.astype(jnp.int32)
    fid = emotion_ids.astype(jnp.int32)
    xu, xf = _gather(user_table, uid, feeling_table, fid)
    return _mlp(xu, xf, W1, b1, W2, b2, W3, b3)


# speedup vs baseline: 1.6814x; 1.6814x over previous
import functools

import jax
import jax.numpy as jnp
from jax import lax
from jax.experimental import pallas as pl
from jax.experimental.pallas import tpu as pltpu
from jax.experimental.pallas import tpu_sc as plsc

B = 16384
EMB_DIM = 32

_info = plsc.get_sparse_core_info()
_NC, _NS = _info.num_cores, _info.num_subcores
_NW = _NC * _NS
_BPW = B // _NW


def _make_gather():
    mesh = plsc.VectorSubcoreMesh(core_axis_name="c", subcore_axis_name="s")

    @functools.partial(
        pl.kernel,
        mesh=mesh,
        compiler_params=pltpu.CompilerParams(use_tc_tiling_on_sc=False),
        out_type=(
            jax.ShapeDtypeStruct((B, EMB_DIM), jnp.float32),
            jax.ShapeDtypeStruct((B, EMB_DIM), jnp.float32),
        ),
        scratch_types=[
            pltpu.VMEM((_BPW,), jnp.int32),
            pltpu.VMEM((_BPW, EMB_DIM), jnp.float32),
            pltpu.VMEM((_BPW,), jnp.int32),
            pltpu.VMEM((_BPW, EMB_DIM), jnp.float32),
            pltpu.SemaphoreType.DMA,
            pltpu.SemaphoreType.DMA,
        ],
    )
    def gather_k(ut_hbm, uid_hbm, ft_hbm, fid_hbm, out_u, out_f,
                 uidx_v, urows_v, fidx_v, frows_v, usem, fsem):
        wid = lax.axis_index("s") * _NC + lax.axis_index("c")
        base = wid * _BPW
        pltpu.sync_copy(uid_hbm.at[pl.ds(base, _BPW)], uidx_v)
        pltpu.sync_copy(fid_hbm.at[pl.ds(base, _BPW)], fidx_v)
        cu = pltpu.async_copy(ut_hbm.at[uidx_v], urows_v, usem)
        cf = pltpu.async_copy(ft_hbm.at[fidx_v], frows_v, fsem)
        cu.wait()
        cf.wait()
        pltpu.sync_copy(urows_v, out_u.at[pl.ds(base, _BPW)])
        pltpu.sync_copy(frows_v, out_f.at[pl.ds(base, _BPW)])

    return gather_k


_gather = _make_gather()

_BM = 1024


def _mlp_body(xu_ref, xf_ref, w1_ref, b1_ref, w2_ref, b2_ref, w3_ref, b3_ref,
              out_ref):
    x = jnp.concatenate([xu_ref[...], xf_ref[...]], axis=1)
    h = jnp.dot(x, w1_ref[...], preferred_element_type=jnp.float32)
    h = jnp.maximum(h + b1_ref[...], 0.0)
    h = jnp.dot(h, w2_ref[...], preferred_element_type=jnp.float32)
    h = jnp.maximum(h + b2_ref[...], 0.0)
    out_ref[...] = (
        jnp.dot(h, w3_ref[...], preferred_element_type=jnp.float32)
        + b3_ref[...]
    )


def _mlp(xu, xf, W1, b1, W2, b2, W3, b3):
    d1, d2, d3 = W1.shape[1], W2.shape[1], W3.shape[1]
    grid = (B // _BM,)

    def full(shape):
        return pl.BlockSpec(shape, lambda i: (0, 0))

    return pl.pallas_call(
        _mlp_body,
        grid=grid,
        in_specs=[
            pl.BlockSpec((_BM, EMB_DIM), lambda i: (i, 0)),
            pl.BlockSpec((_BM, EMB_DIM), lambda i: (i, 0)),
            full(W1.shape),
            full((1, d1)),
            full(W2.shape),
            full((1, d2)),
            full(W3.shape),
            full((1, d3)),
        ],
        out_specs=pl.BlockSpec((_BM, d3), lambda i: (i, 0)),
        out_shape=jax.ShapeDtypeStruct((B, d3), jnp.float32),
    )(xu, xf, W1, b1.reshape(1, d1), W2, b2.reshape(1, d2), W3,
      b3.reshape(1, d3))


def kernel(user_ids, emotion_ids, user_table, feeling_table,
           W1, b1, W2, b2, W3, b3):
    uid = user_ids.astype(jnp.int32)
    fid = emotion_ids.astype(jnp.int32)
    xu, xf = _gather(user_table, uid, feeling_table, fid)
    return _mlp(xu, xf, W1, b1, W2, b2, W3, b3)
